# Initial kernel scaffold; baseline (speedup 1.0000x reference)
import jax
import jax.numpy as jnp
from jax.experimental import pallas as pl


def kernel(x, table, W, b):
    raise NotImplementedError("write your pallas kernel here")


# speedup vs baseline: 2.1592x; 2.1592x over previous
import functools

import jax
import jax.numpy as jnp
from jax import lax
from jax.experimental import pallas as pl
from jax.experimental.pallas import tpu as pltpu
from jax.experimental.pallas import tpu_sc as plsc

B = 16384
LSEQ = 200
D = 64
LABELS = 1000
LANES = 16
NCOL = D // LANES

NC = 2
NS = 16
NW = NC * NS

C = 100
CPS = LSEQ // C
SEG_W = B // NW
CH_W = SEG_W * CPS
GBLK = 64
NBLK = CH_W // GBLK


def _accum_chunk(buf, accs):
    def rbody(r, accs):
        out = list(accs)
        for rr in range(4):
            row = r * 4 + rr
            for cc in range(NCOL):
                out[cc] = out[cc] + buf[row, pl.ds(16 * cc, LANES)]
        return tuple(out)
    return lax.fori_loop(0, C // 4, rbody, accs)


def _sc_pool(x2d, table):
    mesh = plsc.VectorSubcoreMesh(
        core_axis_name="c", subcore_axis_name="s", num_cores=NC,
        num_subcores=NS)

    @functools.partial(
        pl.kernel,
        out_type=jax.ShapeDtypeStruct((B, D), jnp.float32),
        mesh=mesh,
        compiler_params=pltpu.CompilerParams(use_tc_tiling_on_sc=False),
        scratch_types=[
            pltpu.VMEM((GBLK, C), jnp.int32),
            pltpu.VMEM((C, D), jnp.float32),
            pltpu.VMEM((C, D), jnp.float32),
            pltpu.VMEM((SEG_W, D), jnp.float32),
            pltpu.SemaphoreType.DMA,
            pltpu.SemaphoreType.DMA,
        ],
    )
    def k(x_hbm, tab_hbm, out_hbm, idx_v, r0, r1, acc_v, sem0, sem1):
        cid = lax.axis_index("c")
        sid = lax.axis_index("s")
        wid = sid * NC + cid
        chunk_base = wid * CH_W
        seg_base = wid * SEG_W
        scale = jnp.full((LANES,), 1.0 / LSEQ, jnp.float32)

        def block_body(blk, _):
            pltpu.sync_copy(
                x_hbm.at[pl.ds(chunk_base + blk * GBLK, GBLK)], idx_v)

            def seg_body(t, _):
                cp0 = pltpu.async_copy(tab_hbm.at[idx_v.at[2 * t]], r0, sem0)
                cp1 = pltpu.async_copy(
                    tab_hbm.at[idx_v.at[2 * t + 1]], r1, sem1)
                zero = jnp.zeros((LANES,), jnp.float32)
                cp0.wait()
                accs = _accum_chunk(r0, (zero,) * NCOL)
                cp1.wait()
                accs = _accum_chunk(r1, accs)
                seg = blk * (GBLK // CPS) + t
                for cc in range(NCOL):
                    acc_v[seg, pl.ds(16 * cc, LANES)] = accs[cc] * scale
                return 0

            lax.fori_loop(0, GBLK // CPS, seg_body, 0)
            return 0

        lax.fori_loop(0, NBLK, block_body, 0)
        pltpu.sync_copy(acc_v, out_hbm.at[pl.ds(seg_base, SEG_W)])

    return k(x2d, table)


def _tc_head(pooled, W, b2d):
    TB = 1024

    def mm(p_ref, w_ref, b_ref, o_ref):
        acc = lax.dot_general(
            p_ref[...], w_ref[...], (((1,), (1,)), ((), ())),
            preferred_element_type=jnp.float32)
        o_ref[...] = acc + b_ref[...]

    return pl.pallas_call(
        mm,
        grid=(B // TB,),
        in_specs=[
            pl.BlockSpec((TB, D), lambda i: (i, 0)),
            pl.BlockSpec((LABELS, D), lambda i: (0, 0)),
            pl.BlockSpec((1, LABELS), lambda i: (0, 0)),
        ],
        out_specs=pl.BlockSpec((TB, LABELS), lambda i: (i, 0)),
        out_shape=jax.ShapeDtypeStruct((B, LABELS), jnp.float32),
    )(pooled, W, b2d)


def kernel(x, table, W, b):
    x2d = x.reshape(B * LSEQ // C, C)
    pooled = _sc_pool(x2d, table)
    return _tc_head(pooled, W, b.reshape(1, LABELS))


# speedup vs baseline: 2.3595x; 1.0927x over previous
import functools

import jax
import jax.numpy as jnp
from jax import lax
from jax.experimental import pallas as pl
from jax.experimental.pallas import tpu as pltpu
from jax.experimental.pallas import tpu_sc as plsc

B = 16384
LSEQ = 200
D = 64
LABELS = 1000
LANES = 16
NCOL = D // LANES

NC = 2
NS = 16
NW = NC * NS

C = 100
CPS = LSEQ // C
SEG_W = B // NW
CH_W = SEG_W * CPS
GBLK = 64
NBLK = CH_W // GBLK


def _accum_chunk(buf, accs):
    def rbody(r, accs):
        out = list(accs)
        for rr in range(4):
            row = r * 4 + rr
            for cc in range(NCOL):
                out[cc] = out[cc] + buf[row, pl.ds(16 * cc, LANES)]
        return tuple(out)
    return lax.fori_loop(0, C // 4, rbody, accs)


def _sc_pool(x2d, table):
    mesh = plsc.VectorSubcoreMesh(
        core_axis_name="c", subcore_axis_name="s", num_cores=NC,
        num_subcores=NS)

    @functools.partial(
        pl.kernel,
        out_type=jax.ShapeDtypeStruct((B, D), jnp.float32),
        mesh=mesh,
        compiler_params=pltpu.CompilerParams(use_tc_tiling_on_sc=False),
        scratch_types=[
            pltpu.VMEM((GBLK, C), jnp.int32),
            pltpu.VMEM((C, D), jnp.float32),
            pltpu.VMEM((C, D), jnp.float32),
            pltpu.VMEM((C, D), jnp.float32),
            pltpu.VMEM((C, D), jnp.float32),
            pltpu.VMEM((SEG_W, D), jnp.float32),
            pltpu.SemaphoreType.DMA,
            pltpu.SemaphoreType.DMA,
            pltpu.SemaphoreType.DMA,
            pltpu.SemaphoreType.DMA,
        ],
    )
    def k(x_hbm, tab_hbm, out_hbm, idx_v, r0, r1, r2, r3, acc_v,
          sem0, sem1, sem2, sem3):
        cid = lax.axis_index("c")
        sid = lax.axis_index("s")
        wid = sid * NC + cid
        chunk_base = wid * CH_W
        seg_base = wid * SEG_W
        scale = jnp.full((LANES,), 1.0 / LSEQ, jnp.float32)
        bufs = (r0, r1, r2, r3)
        sems = (sem0, sem1, sem2, sem3)

        def block_body(blk, _):
            pltpu.sync_copy(
                x_hbm.at[pl.ds(chunk_base + blk * GBLK, GBLK)], idx_v)

            def grp_body(t, _):
                cps = [
                    pltpu.async_copy(
                        tab_hbm.at[idx_v.at[4 * t + i]], bufs[i], sems[i])
                    for i in range(4)
                ]
                zero = jnp.zeros((LANES,), jnp.float32)
                seg = blk * (GBLK // CPS) + 2 * t
                for half in range(2):
                    cps[2 * half].wait()
                    accs = _accum_chunk(bufs[2 * half], (zero,) * NCOL)
                    cps[2 * half + 1].wait()
                    accs = _accum_chunk(bufs[2 * half + 1], accs)
                    for cc in range(NCOL):
                        acc_v[seg + half, pl.ds(16 * cc, LANES)] = (
                            accs[cc] * scale)
                return 0

            lax.fori_loop(0, GBLK // 4, grp_body, 0)
            return 0

        lax.fori_loop(0, NBLK, block_body, 0)
        pltpu.sync_copy(acc_v, out_hbm.at[pl.ds(seg_base, SEG_W)])

    return k(x2d, table)


def _tc_head(pooled, W, b2d):
    TB = 1024

    def mm(p_ref, w_ref, b_ref, o_ref):
        acc = lax.dot_general(
            p_ref[...], w_ref[...], (((1,), (1,)), ((), ())),
            preferred_element_type=jnp.float32)
        o_ref[...] = acc + b_ref[...]

    return pl.pallas_call(
        mm,
        grid=(B // TB,),
        in_specs=[
            pl.BlockSpec((TB, D), lambda i: (i, 0)),
            pl.BlockSpec((LABELS, D), lambda i: (0, 0)),
            pl.BlockSpec((1, LABELS), lambda i: (0, 0)),
        ],
        out_specs=pl.BlockSpec((TB, LABELS), lambda i: (i, 0)),
        out_shape=jax.ShapeDtypeStruct((B, LABELS), jnp.float32),
    )(pooled, W, b2d)


def kernel(x, table, W, b):
    x2d = x.reshape(B * LSEQ // C, C)
    pooled = _sc_pool(x2d, table)
    return _tc_head(pooled, W, b.reshape(1, LABELS))


# speedup vs baseline: 3.1319x; 1.3274x over previous
import functools

import jax
import jax.numpy as jnp
from jax import lax
from jax.experimental import pallas as pl
from jax.experimental.pallas import tpu as pltpu
from jax.experimental.pallas import tpu_sc as plsc

B = 16384
LSEQ = 200
D = 64
LABELS = 1000
LANES = 16
NCOL = D // LANES

NC = 2
NS = 16
NW = NC * NS

C = 100
CPS = LSEQ // C
SEG_W = B // NW
CH_W = SEG_W * CPS
GBLK = 128
NBLK = CH_W // GBLK
RING = 8
GRPS = GBLK // RING
assert GBLK % RING == 0 and RING % CPS == 0


def _accum_chunk(buf, accs):
    def rbody(r, accs):
        out = list(accs)
        for rr in range(10):
            row = r * 10 + rr
            for cc in range(NCOL):
                out[cc] = out[cc] + buf[row, pl.ds(16 * cc, LANES)]
        return tuple(out)
    return lax.fori_loop(0, C // 10, rbody, accs)


def _sc_pool(x2d, table):
    mesh = plsc.VectorSubcoreMesh(
        core_axis_name="c", subcore_axis_name="s", num_cores=NC,
        num_subcores=NS)

    @functools.partial(
        pl.kernel,
        out_type=jax.ShapeDtypeStruct((B, D), jnp.float32),
        mesh=mesh,
        compiler_params=pltpu.CompilerParams(use_tc_tiling_on_sc=False),
        scratch_types=[
            pltpu.VMEM((GBLK, C), jnp.int32),
            [pltpu.VMEM((C, D), jnp.float32) for _ in range(RING)],
            pltpu.VMEM((SEG_W, D), jnp.float32),
            [pltpu.SemaphoreType.DMA for _ in range(RING)],
        ],
    )
    def k(x_hbm, tab_hbm, out_hbm, idx_v, bufs, acc_v, sems):
        cid = lax.axis_index("c")
        sid = lax.axis_index("s")
        wid = sid * NC + cid
        chunk_base = wid * CH_W
        seg_base = wid * SEG_W
        scale = jnp.full((LANES,), 1.0 / LSEQ, jnp.float32)
        zero = jnp.zeros((LANES,), jnp.float32)

        def start(j, t):
            pltpu.async_copy(tab_hbm.at[idx_v.at[j]], bufs[t], sems[t])

        def wait(j, t):
            pltpu.make_async_copy(
                tab_hbm.at[idx_v.at[j]], bufs[t], sems[t]).wait()

        def block_body(blk, _):
            pltpu.sync_copy(
                x_hbm.at[pl.ds(chunk_base + blk * GBLK, GBLK)], idx_v)
            for t in range(RING):
                start(t, t)

            seg0 = blk * (GBLK // CPS)

            def drain(g, issue_next):
                accs = None
                for t in range(RING):
                    j = g * RING + t
                    wait(j, t)
                    if t % CPS == 0:
                        accs = _accum_chunk(bufs[t], (zero,) * NCOL)
                    else:
                        accs = _accum_chunk(bufs[t], accs)
                        seg = seg0 + g * (RING // CPS) + t // CPS
                        for cc in range(NCOL):
                            acc_v[seg, pl.ds(16 * cc, LANES)] = (
                                accs[cc] * scale)
                    if issue_next:
                        start(j + RING, t)

            def grp_body(g, _):
                drain(g, True)
                return 0

            lax.fori_loop(0, GRPS - 1, grp_body, 0)
            drain(GRPS - 1, False)
            return 0

        lax.fori_loop(0, NBLK, block_body, 0)
        pltpu.sync_copy(acc_v, out_hbm.at[pl.ds(seg_base, SEG_W)])

    return k(x2d, table)


def _tc_head(pooled, W, b2d):
    TB = 1024

    def mm(p_ref, w_ref, b_ref, o_ref):
        acc = lax.dot_general(
            p_ref[...], w_ref[...], (((1,), (1,)), ((), ())),
            preferred_element_type=jnp.float32)
        o_ref[...] = acc + b_ref[...]

    return pl.pallas_call(
        mm,
        grid=(B // TB,),
        in_specs=[
            pl.BlockSpec((TB, D), lambda i: (i, 0)),
            pl.BlockSpec((LABELS, D), lambda i: (0, 0)),
            pl.BlockSpec((1, LABELS), lambda i: (0, 0)),
        ],
        out_specs=pl.BlockSpec((TB, LABELS), lambda i: (i, 0)),
        out_shape=jax.ShapeDtypeStruct((B, LABELS), jnp.float32),
    )(pooled, W, b2d)


def kernel(x, table, W, b):
    x2d = x.reshape(B * LSEQ // C, C)
    pooled = _sc_pool(x2d, table)
    return _tc_head(pooled, W, b.reshape(1, LABELS))


# speedup vs baseline: 4.0786x; 1.3023x over previous
import functools

import jax
import jax.numpy as jnp
from jax import lax
from jax.experimental import pallas as pl
from jax.experimental.pallas import tpu as pltpu
from jax.experimental.pallas import tpu_sc as plsc
from jax.experimental import layout as jlayout

B = 16384
LSEQ = 200
D = 64
LABELS = 1000
LANES = 16
NCOL = D // LANES

NC = 2
NS = 16
NW = NC * NS

C = 100
CPS = LSEQ // C
SEG_W = B // NW
CH_W = SEG_W * CPS
GBLK = 128
NBLK = CH_W // GBLK
RING = 8
GRPS = GBLK // RING
assert GBLK % RING == 0 and RING % CPS == 0


def _accum_chunk(buf, accs):
    def rbody(r, accs):
        out = list(accs)
        for rr in range(10):
            row = r * 10 + rr
            for cc in range(NCOL):
                out[cc] = out[cc] + buf[row, pl.ds(16 * cc, LANES)]
        return tuple(out)
    return lax.fori_loop(0, C // 10, rbody, accs)


def _sc_pool(x2d, table):
    mesh = plsc.VectorSubcoreMesh(
        core_axis_name="c", subcore_axis_name="s", num_cores=NC,
        num_subcores=NS)

    @functools.partial(
        pl.kernel,
        out_type=jax.ShapeDtypeStruct((B, D), jnp.float32),
        mesh=mesh,
        compiler_params=pltpu.CompilerParams(use_tc_tiling_on_sc=False),
        scratch_types=[
            pltpu.VMEM((GBLK, C), jnp.int32),
            [pltpu.VMEM((C, D), jnp.float32) for _ in range(RING)],
            pltpu.VMEM((SEG_W, D), jnp.float32),
            [pltpu.SemaphoreType.DMA for _ in range(RING)],
        ],
    )
    def k(x_hbm, tab_hbm, out_hbm, idx_v, bufs, acc_v, sems):
        cid = lax.axis_index("c")
        sid = lax.axis_index("s")
        wid = sid * NC + cid
        chunk_base = wid * CH_W
        seg_base = wid * SEG_W
        scale = jnp.full((LANES,), 1.0 / LSEQ, jnp.float32)
        zero = jnp.zeros((LANES,), jnp.float32)

        def start(j, t):
            pltpu.async_copy(tab_hbm.at[idx_v.at[j]], bufs[t], sems[t])

        def wait(j, t):
            pltpu.make_async_copy(
                tab_hbm.at[idx_v.at[j]], bufs[t], sems[t]).wait()

        def block_body(blk, _):
            pltpu.sync_copy(
                x_hbm.at[pl.ds(chunk_base + blk * GBLK, GBLK)], idx_v)
            for t in range(RING):
                start(t, t)

            seg0 = blk * (GBLK // CPS)

            def drain(g, issue_next):
                accs = None
                for t in range(RING):
                    j = g * RING + t
                    wait(j, t)
                    if t % CPS == 0:
                        accs = _accum_chunk(bufs[t], (zero,) * NCOL)
                    else:
                        accs = _accum_chunk(bufs[t], accs)
                        seg = seg0 + g * (RING // CPS) + t // CPS
                        for cc in range(NCOL):
                            acc_v[seg, pl.ds(16 * cc, LANES)] = (
                                accs[cc] * scale)
                    if issue_next:
                        start(j + RING, t)

            def grp_body(g, _):
                drain(g, True)
                return 0

            lax.fori_loop(0, GRPS - 1, grp_body, 0)
            drain(GRPS - 1, False)
            return 0

        lax.fori_loop(0, NBLK, block_body, 0)
        pltpu.sync_copy(acc_v, out_hbm.at[pl.ds(seg_base, SEG_W)])

    return k(x2d, table)


def _tc_head(pooled, W, b2d):
    TB = 1024

    def mm(p_ref, w_ref, b_ref, o_ref):
        acc = lax.dot_general(
            p_ref[...], w_ref[...], (((1,), (1,)), ((), ())),
            preferred_element_type=jnp.float32)
        o_ref[...] = acc + b_ref[...]

    return pl.pallas_call(
        mm,
        grid=(B // TB,),
        in_specs=[
            pl.BlockSpec((TB, D), lambda i: (i, 0)),
            pl.BlockSpec((LABELS, D), lambda i: (0, 0)),
            pl.BlockSpec((1, LABELS), lambda i: (0, 0)),
        ],
        out_specs=pl.BlockSpec((TB, LABELS), lambda i: (i, 0)),
        out_shape=jax.ShapeDtypeStruct((B, LABELS), jnp.float32),
    )(pooled, W, b2d)


def kernel(x, table, W, b):
    x2d = x.reshape(B * LSEQ // C, C)
    table_lin = jlayout.with_layout_constraint(
        table, jlayout.Layout((0, 1), tiling=()))
    pooled = _sc_pool(x2d, table_lin)
    return _tc_head(pooled, W, b.reshape(1, LABELS))
